# baseline (speedup 1.0000x reference)
import functools

import jax
import jax.numpy as jnp
from jax.experimental import pallas as pl
from jax.experimental.pallas import tpu as pltpu

_EPS = 1e-5


def _fused_msg_body(p_ref, scale_ref, shift_ref, w2_ref, t_ref):
    r = jnp.maximum(p_ref[...] * scale_ref[...] + shift_ref[...], 0.0)
    t_ref[...] = jnp.dot(r, w2_ref[...], preferred_element_type=jnp.float32)


def _fused_msg(P, scale, shift, w2, blk=2000):
    E, D = P.shape
    Dout = w2.shape[1]
    nblk = (E + blk - 1) // blk
    Epad = nblk * blk
    if Epad != E:
        P = jnp.pad(P, ((0, Epad - E), (0, 0)))
    out = pl.pallas_call(
        _fused_msg_body,
        grid=(nblk,),
        in_specs=[
            pl.BlockSpec((blk, D), lambda i: (i, 0)),
            pl.BlockSpec((1, D), lambda i: (0, 0)),
            pl.BlockSpec((1, D), lambda i: (0, 0)),
            pl.BlockSpec((D, Dout), lambda i: (0, 0)),
        ],
        out_specs=pl.BlockSpec((blk, Dout), lambda i: (i, 0)),
        out_shape=jax.ShapeDtypeStruct((Epad, Dout), jnp.float32),
    )(P, scale[None, :], shift[None, :], w2)
    return out[:E]


def _bn_coeffs(mean, var, g, bt):
    inv = g * jax.lax.rsqrt(var + _EPS)
    return inv, bt - mean * inv


def _bn_stats(x):
    mu = jnp.mean(x, axis=0)
    var = jnp.mean(x * x, axis=0) - mu * mu
    return mu, var


def kernel(x, edge_index, params):
    n_nodes = x.shape[0]
    src = edge_index[0]
    dst = edge_index[1]
    emb = params['lin_in_w'].shape[1]

    h = x @ params['lin_in_w'] + params['lin_in_b']

    for lp in params['layers']:
        mp, up = lp['msg'], lp['upd']
        A = h @ mp['w1'][:emb]
        B = h @ mp['w1'][emb:]
        P = A[dst] + B[src]
        mu1, var1 = _bn_stats(P)
        sc1, sh1 = _bn_coeffs(mu1, var1, mp['g1'], mp['bt1'])
        T = _fused_msg(P, sc1, sh1, mp['w2'])
        mu2, var2 = _bn_stats(T)
        sc2, sh2 = _bn_coeffs(mu2, var2, mp['g2'], mp['bt2'])
        M = jnp.maximum(T * sc2 + sh2, 0.0)
        agg = jax.ops.segment_max(M, dst, num_segments=n_nodes)
        agg = jnp.where(jnp.isneginf(agg), 0.0, agg)
        U = h @ up['w1'][:emb] + agg @ up['w1'][emb:]
        mu3, var3 = _bn_stats(U)
        sc3, sh3 = _bn_coeffs(mu3, var3, up['g1'], up['bt1'])
        u1 = jnp.maximum(U * sc3 + sh3, 0.0)
        V = u1 @ up['w2']
        mu4, var4 = _bn_stats(V)
        sc4, sh4 = _bn_coeffs(mu4, var4, up['g2'], up['bt2'])
        h = h + jnp.maximum(V * sc4 + sh4, 0.0)

    C = h @ params['mlp_w1'][:emb]
    D = h @ params['mlp_w1'][emb:]
    he = jnp.maximum(C[src] + D[dst] + params['mlp_b1'], 0.0)
    he = he @ params['mlp_w2'] + params['mlp_b2']
    he = he[:, 0]
    Emat = jnp.zeros((n_nodes, n_nodes), dtype=h.dtype).at[src, dst].add(he)
    return Emat
